# baseline (speedup 1.0000x reference)
import functools

import jax
import jax.numpy as jnp
from jax import lax
from jax.experimental import pallas as pl
from jax.experimental.pallas import tpu as pltpu
from jax.experimental.pallas import tpu_sc as plsc

N = 10000
E = 160000
D = 256
H = 512
G = 64
KHOP = 3

CF = 128
DW = 128
NPAD = 10240
ECH = 128
NW = 32
EPAD = 163840
EPT = EPAD // NW
NITER = EPT // ECH
RPS = NPAD // 16
ZR = RPS // 4

BN = 400
NBLK = N // BN

_sc_mesh = plsc.VectorSubcoreMesh(core_axis_name="c", subcore_axis_name="s")


@functools.partial(
    pl.kernel,
    out_type=jax.ShapeDtypeStruct((2 * NPAD, DW), jnp.float32),
    mesh=_sc_mesh,
    scratch_types=[
        pltpu.VMEM((NITER, ECH), jnp.int32),
        pltpu.VMEM((ECH, DW), jnp.float32),
        pltpu.VMEM((ECH, DW), jnp.float32),
        pltpu.VMEM_SHARED((NPAD, DW), jnp.float32),
        pltpu.SemaphoreType.DMA,
    ],
)
def _deg_kernel(col_hbm, out_hbm, cidx_all, ones_v, zero_v, acc_sh, sems):
    cid = lax.axis_index("c")
    sid = lax.axis_index("s")
    w = sid * 2 + cid

    pltpu.sync_copy(col_hbm.at[pl.ds(w * NITER, NITER)], cidx_all)

    @pl.loop(0, ECH)
    def _(r):
        for j in range(DW // 16):
            ones_v[r, pl.ds(j * 16, 16)] = jnp.full((16,), 1.0, jnp.float32)

    @pl.loop(0, ECH)
    def _(r):
        for j in range(DW // 16):
            zero_v[r, pl.ds(j * 16, 16)] = jnp.zeros((16,), jnp.float32)

    @pl.loop(0, RPS // ECH)
    def _(b):
        pltpu.sync_copy(zero_v, acc_sh.at[pl.ds(sid * RPS + b * ECH, ECH)])

    plsc.subcore_barrier()

    for t in range(NITER):
        pltpu.sync_copy(ones_v, acc_sh.at[cidx_all.at[t]], add=True)

    plsc.subcore_barrier()
    pltpu.sync_copy(acc_sh.at[pl.ds(sid * RPS, RPS)],
                    out_hbm.at[pl.ds(cid * NPAD + sid * RPS, RPS)])


def _make_prop(nf):
    @functools.partial(
        pl.kernel,
        out_type=jax.ShapeDtypeStruct((nf * 2 * NPAD, CF), jnp.float32),
        mesh=_sc_mesh,
        scratch_types=[
            pltpu.VMEM((NITER, ECH), jnp.int32),
            pltpu.VMEM((NITER, ECH), jnp.int32),
            pltpu.VMEM((ECH, CF), jnp.float32),
            pltpu.VMEM((ECH, CF), jnp.float32),
            pltpu.VMEM_SHARED((NPAD, CF), jnp.float32),
            pltpu.SemaphoreType.DMA,
            pltpu.SemaphoreType.DMA,
        ],
    )
    def prop(*args):
        us = args[:nf]
        row_hbm, col_hbm, out_hbm = args[nf], args[nf + 1], args[nf + 2]
        ridx_all, cidx_all, buf0, buf1, acc_sh, semg, sems = args[nf + 3:]
        cid = lax.axis_index("c")
        sid = lax.axis_index("s")
        w = sid * 2 + cid
        bufs = (buf0, buf1)

        pltpu.sync_copy(row_hbm.at[pl.ds(w * NITER, NITER)], ridx_all)
        pltpu.sync_copy(col_hbm.at[pl.ds(w * NITER, NITER)], cidx_all)

        for f in range(nf):
            @pl.loop(0, ECH)
            def _(r):
                for j in range(CF // 16):
                    buf0[r, pl.ds(j * 16, 16)] = jnp.zeros((16,), jnp.float32)

            for q in range(RPS // ECH):
                pltpu.sync_copy(buf0,
                                acc_sh.at[pl.ds(sid * RPS + q * ECH, ECH)])
            plsc.subcore_barrier()

            def gather(t, f=f):
                return pltpu.async_copy(us[f].at[ridx_all.at[t]],
                                        bufs[t % 2], semg)

            def scatter(t):
                return pltpu.async_copy(bufs[t % 2],
                                        acc_sh.at[cidx_all.at[t]],
                                        sems, add=True)

            dg_cur = gather(0)
            ds_prev = None
            for t in range(NITER):
                dg_cur.wait()
                if ds_prev is not None:
                    ds_prev.wait()
                if t + 1 < NITER:
                    dg_cur = gather(t + 1)
                ds_prev = scatter(t)
            ds_prev.wait()

            plsc.subcore_barrier()
            pltpu.sync_copy(
                acc_sh.at[pl.ds(sid * RPS, RPS)],
                out_hbm.at[pl.ds((f * 2 + cid) * NPAD + sid * RPS, RPS)])
            plsc.subcore_barrier()

    return prop


_prop2_kernel = _make_prop(2)
_prop4_kernel = _make_prop(4)


def _dinv_from(degp_ref):
    degs = degp_ref[0] + degp_ref[1]
    deg = degs[:, :1]
    return jnp.where(deg > 0, lax.rsqrt(deg), 0.0)


def _prep_body(degp_ref, x_ref, w_ref, out_ref, u0_ref, u1_ref):
    dinv = _dinv_from(degp_ref)
    x = x_ref[...]
    out_ref[...] = jnp.dot(x, w_ref[...], preferred_element_type=jnp.float32)
    u = dinv * x
    u0_ref[...] = u[:, :CF]
    u1_ref[...] = u[:, CF:]


def _merge_body(nf, degp_ref, outp_ref, w_ref, *rest):
    s_refs = rest[:nf]
    out_ref = rest[nf]
    u_refs = rest[nf + 1:]
    dinv = _dinv_from(degp_ref)
    ssum = jnp.concatenate([s[0] + s[1] for s in s_refs], axis=1)
    h = dinv * ssum
    out_ref[...] = outp_ref[...] + jnp.dot(
        h, w_ref[...], preferred_element_type=jnp.float32)
    for j in range(nf):
        u_refs[j][...] = dinv * h[:, j * CF:(j + 1) * CF]


def _final1_body(degp_ref, outp_ref, w3_ref, b_ref, w10_ref, s0_ref, s1_ref,
                 out2_ref, u0_ref, u1_ref, u2_ref, u3_ref):
    dinv = _dinv_from(degp_ref)
    ssum = jnp.concatenate([s0_ref[0] + s0_ref[1], s1_ref[0] + s1_ref[1]],
                           axis=1)
    h = dinv * ssum
    y = outp_ref[...] + jnp.dot(
        h, w3_ref[...], preferred_element_type=jnp.float32) + b_ref[0:1, :]
    x2 = jnp.where(y >= 0, y, 0.25 * y)
    out2_ref[...] = jnp.dot(x2, w10_ref[...],
                            preferred_element_type=jnp.float32)
    u = dinv * x2
    for j, ur in enumerate((u0_ref, u1_ref, u2_ref, u3_ref)):
        ur[...] = u[:, j * CF:(j + 1) * CF]


def _final2_body(degp_ref, outp_ref, w3_ref, b_ref, wout_ref,
                 bat_ref, s0_ref, s1_ref, s2_ref, s3_ref, pooled_ref, res_ref):
    i = pl.program_id(0)
    dinv = _dinv_from(degp_ref)
    ssum = jnp.concatenate(
        [s[0] + s[1] for s in (s0_ref, s1_ref, s2_ref, s3_ref)], axis=1)
    h = dinv * ssum
    y = outp_ref[...] + jnp.dot(
        h, w3_ref[...], preferred_element_type=jnp.float32) + b_ref[0:1, :]
    x3 = jnp.where(y >= 0, y, 0.25 * y)
    ids = bat_ref[0]
    gid = lax.broadcasted_iota(jnp.int32, (G, BN), 0)
    mask = (gid == ids).astype(jnp.float32)
    part = jnp.dot(mask, x3, preferred_element_type=jnp.float32,
                   precision=lax.Precision.HIGHEST)

    @pl.when(i == 0)
    def _():
        pooled_ref[...] = jnp.zeros((G, H), jnp.float32)

    pooled_ref[...] += part

    @pl.when(i == NBLK - 1)
    def _():
        res_ref[...] = jnp.dot(pooled_ref[...], wout_ref[...],
                               preferred_element_type=jnp.float32)


def _sblock(i):
    return (0, i, 0)


_DEG_SPEC = pl.BlockSpec((2, BN, DW), _sblock)
_S_SPEC = pl.BlockSpec((2, BN, CF), _sblock)
_H_SPEC = pl.BlockSpec((BN, H), lambda i: (i, 0))
_U_SPEC = pl.BlockSpec((BN, CF), lambda i: (i, 0))
_H_OUT = jax.ShapeDtypeStruct((N, H), jnp.float32)
_U_OUT = jax.ShapeDtypeStruct((N, CF), jnp.float32)


def _full(shape):
    return pl.BlockSpec(shape, lambda i: tuple(0 for _ in shape))


_prep = pl.pallas_call(
    _prep_body,
    grid=(NBLK,),
    in_specs=[_DEG_SPEC, pl.BlockSpec((BN, D), lambda i: (i, 0)),
              pl.BlockSpec((D, H), lambda i: (0, 0))],
    out_specs=[_H_SPEC, _U_SPEC, _U_SPEC],
    out_shape=[_H_OUT, _U_OUT, _U_OUT],
)

def _wspec(din, k):
    return pl.BlockSpec((din, H), lambda i, _k=k: (_k, 0))


_merge2_k = {
    k: pl.pallas_call(
        functools.partial(_merge_body, 2),
        grid=(NBLK,),
        in_specs=[_DEG_SPEC, _H_SPEC, _wspec(D, k), _S_SPEC, _S_SPEC],
        out_specs=[_H_SPEC, _U_SPEC, _U_SPEC],
        out_shape=[_H_OUT, _U_OUT, _U_OUT],
    ) for k in (1, 2)
}

_merge4_k = {
    k: pl.pallas_call(
        functools.partial(_merge_body, 4),
        grid=(NBLK,),
        in_specs=[_DEG_SPEC, _H_SPEC, _wspec(H, k),
                  _S_SPEC, _S_SPEC, _S_SPEC, _S_SPEC],
        out_specs=[_H_SPEC, _U_SPEC, _U_SPEC, _U_SPEC, _U_SPEC],
        out_shape=[_H_OUT, _U_OUT, _U_OUT, _U_OUT, _U_OUT],
    ) for k in (1, 2)
}

_final1 = pl.pallas_call(
    _final1_body,
    grid=(NBLK,),
    in_specs=[_DEG_SPEC, _H_SPEC, _wspec(D, 3), _full((8, H)),
              _wspec(H, 0), _S_SPEC, _S_SPEC],
    out_specs=[_H_SPEC, _U_SPEC, _U_SPEC, _U_SPEC, _U_SPEC],
    out_shape=[_H_OUT, _U_OUT, _U_OUT, _U_OUT, _U_OUT],
)

_final2 = pl.pallas_call(
    _final2_body,
    grid=(NBLK,),
    in_specs=[_DEG_SPEC, _H_SPEC, _wspec(H, 3), _full((8, H)),
              _full((H, CF)),
              pl.BlockSpec((1, 1, BN), lambda i: (i, 0, 0)),
              _S_SPEC, _S_SPEC, _S_SPEC, _S_SPEC],
    out_specs=[pl.BlockSpec((G, H), lambda i: (0, 0)),
               pl.BlockSpec((G, CF), lambda i: (0, 0))],
    out_shape=[jax.ShapeDtypeStruct((G, H), jnp.float32),
               jax.ShapeDtypeStruct((G, CF), jnp.float32)],
)


def _prop2(u0, u1, row_p, col_p):
    s = _prop2_kernel(u0, u1, row_p, col_p).reshape(2, 2, NPAD, CF)
    return s[0], s[1]


def _prop4(v0, v1, v2, v3, row_p, col_p):
    s = _prop4_kernel(v0, v1, v2, v3, row_p, col_p).reshape(4, 2, NPAD, CF)
    return s[0], s[1], s[2], s[3]


def kernel(x, edge_index, batch, W0, b0, W1, b1, Wout, bout):
    (x, edge_index, batch, W0, b0, W1, b1, Wout, bout) = (
        lax.optimization_barrier(
            (x, edge_index, batch, W0, b0, W1, b1, Wout, bout)))
    row = edge_index[0]
    col = edge_index[1]
    pad = EPAD - E
    row_p = jnp.concatenate(
        [row, jnp.zeros((pad,), jnp.int32)]).reshape(NW * NITER, ECH)
    col_p = jnp.concatenate(
        [col, jnp.full((pad,), N, jnp.int32)]).reshape(NW * NITER, ECH)
    batch2d = batch.reshape(NBLK, 1, BN)
    b0r = jnp.broadcast_to(b0.reshape(1, H), (8, H))
    b1r = jnp.broadcast_to(b1.reshape(1, H), (8, H))
    woutp = jnp.pad(Wout, ((0, 0), (0, CF - 1)))
    W0r = W0.reshape(4 * D, H)
    W1r = W1.reshape(4 * H, H)

    degp = _deg_kernel(col_p).reshape(2, NPAD, DW)

    out, u0, u1 = _prep(degp, x, W0r)
    for k in (1, 2):
        s0, s1 = _prop2(u0, u1, row_p, col_p)
        out, u0, u1 = _merge2_k[k](degp, out, W0r, s0, s1)
    s0, s1 = _prop2(u0, u1, row_p, col_p)
    out2, v0, v1, v2, v3 = _final1(degp, out, W0r, b0r, W1r, s0, s1)

    for k in (1, 2):
        t0, t1, t2, t3 = _prop4(v0, v1, v2, v3, row_p, col_p)
        out2, v0, v1, v2, v3 = _merge4_k[k](degp, out2, W1r, t0, t1, t2, t3)
    t0, t1, t2, t3 = _prop4(v0, v1, v2, v3, row_p, col_p)

    _, res = _final2(degp, out2, W1r, b1r, woutp, batch2d,
                     t0, t1, t2, t3)
    return res[:, :1] + bout
